# baseline (speedup 1.0000x reference)
import functools

import jax
import jax.numpy as jnp
from jax import lax
from jax.experimental import pallas as pl
from jax.experimental.pallas import tpu as pltpu
from jax.experimental.pallas import tpu_sc as plsc

B, A, D, KM1, K_TOP = 64, 4096, 256, 64, 64
B_TILE = 4

_NC, _NS, _L = 2, 16, 16
_NW = _NC * _NS
_RPW = B // _NW
_NV = A // _L


def _score_body(x_ref, w_ref, b_ref, o_ref):
    x = x_ref[...].reshape(B_TILE * A, D)
    logits_t = jax.lax.dot_general(
        w_ref[...], x, (((0,), (1,)), ((), ())),
        preferred_element_type=jnp.float32)
    s = jax.nn.sigmoid(logits_t + b_ref[...])
    o_ref[...] = (s.sum(axis=0) * (1.0 / KM1) - 0.5).reshape(B_TILE, 1, A)


def _sc_select_body(scores_hbm, out_hbm, row_v, bits_v, act_v, b16_v, cand_v):
    wid = lax.axis_index("s") * _NC + lax.axis_index("c")
    iota16 = lax.iota(jnp.int32, _L)
    for rr in range(_RPW):
        r = wid * _RPW + rr
        pltpu.sync_copy(scores_hbm.at[r, 0], row_v)

        def p1(v, carry):
            sla = pl.ds((2 * v) * _L, _L)
            slb = pl.ds((2 * v + 1) * _L, _L)
            ba = lax.bitcast_convert_type(row_v[sla], jnp.int32) & 0x7FFFFFFF
            bb = lax.bitcast_convert_type(row_v[slb], jnp.int32) & 0x7FFFFFFF
            bits_v[sla] = ba
            bits_v[slb] = bb
            b16_v[sla] = ba >> 15
            b16_v[slb] = bb >> 15
            return carry
        lax.fori_loop(0, _NV // 2, p1, jnp.int32(0))

        def aiter(_, lohi):
            lo, hi = lohi
            mid = lo + ((hi - lo) >> 1)
            midv = jnp.broadcast_to(mid, (_L,))

            def inner(v, cnt):
                c0, c1 = cnt
                for u in range(8):
                    m = b16_v[pl.ds((v * 8 + u) * _L, _L)] >= midv
                    p = plsc.all_reduce_population_count(m)
                    if u % 2 == 0:
                        c0 = c0 + p
                    else:
                        c1 = c1 + p
                return c0, c1

            c0, c1 = lax.fori_loop(
                0, _NV // 8, inner,
                (jnp.zeros((_L,), jnp.int32), jnp.zeros((_L,), jnp.int32)))
            total = jnp.max(c0 + c1)
            ge = total >= K_TOP
            return jnp.where(ge, mid, lo), jnp.where(ge, hi, mid)

        t16, _ = lax.fori_loop(
            0, 15, aiter, (jnp.int32(0), jnp.int32(0x7E01)))

        t16v = jnp.broadcast_to(t16, (_L,))

        def cpass(v, c):
            off, ngt = c
            b = bits_v[pl.ds(v * _L, _L)]
            hb = b >> 15
            m_eq = hb == t16v
            m_gt = hb > t16v
            eqi = m_eq.astype(jnp.int32)
            exc = plsc.cumsum(eqi) - eqi
            plsc.store_scatter(cand_v, [off + exc], b, mask=m_eq)
            return (off + plsc.all_reduce_population_count(m_eq),
                    ngt + plsc.all_reduce_population_count(m_gt))

        off, ngt16 = lax.fori_loop(
            0, _NV, cpass,
            (jnp.zeros((_L,), jnp.int32), jnp.zeros((_L,), jnp.int32)))
        plsc.store_scatter(cand_v, [off + iota16],
                           jnp.full((_L,), -1, jnp.int32))
        m_cnt = jnp.max(off)
        n_gt16 = jnp.max(ngt16)
        nv_cand = m_cnt // _L + 1

        def biter(_, lohi):
            lo, hi = lohi
            mid = lo + ((hi - lo) >> 1)
            midv = jnp.broadcast_to(mid, (_L,))

            def inner(v, cnt):
                m = cand_v[pl.ds(v * _L, _L)] >= midv
                return cnt + plsc.all_reduce_population_count(m)

            cnt = lax.fori_loop(0, nv_cand, inner, jnp.zeros((_L,), jnp.int32))
            ge = (n_gt16 + jnp.max(cnt)) >= K_TOP
            return jnp.where(ge, mid, lo), jnp.where(ge, hi, mid)

        t_s, _ = lax.fori_loop(
            0, 15, biter, (t16 << 15, (t16 + 1) << 15))
        t = jnp.broadcast_to(t_s, (_L,))

        def p3(v, c):
            n, z = c
            b = bits_v[pl.ds(v * _L, _L)]
            m = b > t
            n = n + plsc.all_reduce_population_count(m)
            z = z + jnp.where(m, lax.bitcast_convert_type(b, jnp.float32), 0.0)
            return n, z

        n_gt, z_vec = lax.fori_loop(
            0, _NV, p3,
            (jnp.zeros((_L,), jnp.int32), jnp.zeros((_L,), jnp.float32)))
        need = K_TOP - n_gt
        t_f = lax.bitcast_convert_type(t, jnp.float32)
        z_sum = jnp.broadcast_to(jnp.sum(z_vec), (_L,))
        z = z_sum + need.astype(jnp.float32) * t_f
        inv = 1.0 / (z + 1e-8)

        def p4(v, carry_eq):
            sl = pl.ds(v * _L, _L)
            b = bits_v[sl]
            s = row_v[sl]
            gt = b > t
            eq = b == t
            eqi = eq.astype(jnp.int32)
            rank = carry_eq + (plsc.cumsum(eqi) - eqi)
            mask = gt | (eq & (rank < need))
            act_v[sl] = jnp.where(mask, s * inv, 0.0)
            return carry_eq + plsc.all_reduce_population_count(eq)

        lax.fori_loop(0, _NV, p4, jnp.zeros((_L,), jnp.int32))
        pltpu.sync_copy(act_v, out_hbm.at[r])


@functools.cache
def _make_sc_select():
    return pl.kernel(
        _sc_select_body,
        mesh=plsc.VectorSubcoreMesh(core_axis_name="c", subcore_axis_name="s"),
        out_type=jax.ShapeDtypeStruct((B, A), jnp.float32),
        scratch_types=[
            pltpu.VMEM((A,), jnp.float32),
            pltpu.VMEM((A,), jnp.int32),
            pltpu.VMEM((A,), jnp.float32),
            pltpu.VMEM((A,), jnp.int32),
            pltpu.VMEM((A + _L,), jnp.int32),
        ],
        compiler_params=pltpu.CompilerParams(needs_layout_passes=False),
    )


@jax.jit
def kernel(signal_features, W, b):
    scores = pl.pallas_call(
        _score_body,
        grid=(B // B_TILE,),
        in_specs=[
            pl.BlockSpec((B_TILE, A, D), lambda i: (i, 0, 0)),
            pl.BlockSpec((D, KM1), lambda i: (0, 0)),
            pl.BlockSpec((KM1, 1), lambda i: (0, 0)),
        ],
        out_specs=pl.BlockSpec((B_TILE, 1, A), lambda i: (i, 0, 0)),
        out_shape=jax.ShapeDtypeStruct((B, 1, A), jnp.float32),
    )(signal_features, W, b.reshape(KM1, 1))
    return _make_sc_select()(scores)


# speedup vs baseline: 1.0736x; 1.0736x over previous
import functools

import jax
import jax.numpy as jnp
from jax import lax
from jax.experimental import pallas as pl
from jax.experimental.pallas import tpu as pltpu
from jax.experimental.pallas import tpu_sc as plsc

B, A, D, KM1, K_TOP = 64, 4096, 256, 64, 64
B_TILE = 4

_NC, _NS, _L = 2, 16, 16
_NW = _NC * _NS
_RPW = B // _NW
_NV = A // _L


def _score_body(x_ref, w_ref, b_ref, o_ref):
    x = x_ref[...].reshape(B_TILE * A, D)
    logits_t = jax.lax.dot_general(
        w_ref[...], x, (((0,), (1,)), ((), ())),
        preferred_element_type=jnp.float32)
    s = jax.nn.sigmoid(logits_t + b_ref[...])
    o_ref[...] = (s.sum(axis=0) * (1.0 / KM1) - 0.5).reshape(B_TILE, 1, A)


def _sc_select_body(scores_hbm, out_hbm, row_v, bits_v, act_v):
    wid = lax.axis_index("s") * _NC + lax.axis_index("c")
    for rr in range(_RPW):
        r = wid * _RPW + rr
        pltpu.sync_copy(scores_hbm.at[r, 0], row_v)

        def p1(v, carry):
            for u in range(4):
                sl = pl.ds((v * 4 + u) * _L, _L)
                bits_v[sl] = (lax.bitcast_convert_type(row_v[sl], jnp.int32)
                              & 0x7FFFFFFF)
            return carry
        lax.fori_loop(0, _NV // 4, p1, jnp.int32(0))

        def citer(_, lohi):
            lo, hi = lohi
            mid = lo + ((hi - lo) >> 1)

            def inner(v, cnt):
                c = list(cnt)
                for u in range(16):
                    m = bits_v[pl.ds((v * 16 + u) * _L, _L)] >= mid
                    p = plsc.all_reduce_population_count(m)
                    c[u % 4] = c[u % 4] + p
                return tuple(c)

            cs = lax.fori_loop(
                0, _NV // 16, inner,
                tuple(jnp.zeros((_L,), jnp.int32) for _ in range(4)))
            ge = (cs[0] + cs[1] + cs[2] + cs[3]) >= K_TOP
            return jnp.where(ge, mid, lo), jnp.where(ge, hi, mid)

        t, _hi = lax.fori_loop(
            0, 30, citer,
            (jnp.zeros((_L,), jnp.int32),
             jnp.full((_L,), 0x3F000001, jnp.int32)))

        def p3(v, c):
            n0, n1, z0, z1 = c
            for u in range(4):
                b = bits_v[pl.ds((v * 4 + u) * _L, _L)]
                m = b > t
                p = plsc.all_reduce_population_count(m)
                zz = jnp.where(m, lax.bitcast_convert_type(b, jnp.float32), 0.0)
                if u % 2 == 0:
                    n0, z0 = n0 + p, z0 + zz
                else:
                    n1, z1 = n1 + p, z1 + zz
            return n0, n1, z0, z1

        n0, n1, z0, z1 = lax.fori_loop(
            0, _NV // 4, p3,
            (jnp.zeros((_L,), jnp.int32), jnp.zeros((_L,), jnp.int32),
             jnp.zeros((_L,), jnp.float32), jnp.zeros((_L,), jnp.float32)))
        n_gt = n0 + n1
        need = K_TOP - n_gt
        t_f = lax.bitcast_convert_type(t, jnp.float32)
        z_sum = jnp.broadcast_to(jnp.sum(z0 + z1), (_L,))
        z = z_sum + need.astype(jnp.float32) * t_f
        inv = 1.0 / (z + 1e-8)

        def p4(v, carry_eq):
            ce = carry_eq
            for u in range(2):
                sl = pl.ds((v * 2 + u) * _L, _L)
                b = bits_v[sl]
                s = row_v[sl]
                gt = b > t
                eq = b == t
                eqi = eq.astype(jnp.int32)
                rank = ce + (plsc.cumsum(eqi) - eqi)
                mask = gt | (eq & (rank < need))
                act_v[sl] = jnp.where(mask, s * inv, 0.0)
                ce = ce + plsc.all_reduce_population_count(eq)
            return ce

        lax.fori_loop(0, _NV // 2, p4, jnp.zeros((_L,), jnp.int32))
        pltpu.sync_copy(act_v, out_hbm.at[r])


@functools.cache
def _make_sc_select():
    return pl.kernel(
        _sc_select_body,
        mesh=plsc.VectorSubcoreMesh(core_axis_name="c", subcore_axis_name="s"),
        out_type=jax.ShapeDtypeStruct((B, A), jnp.float32),
        scratch_types=[
            pltpu.VMEM((A,), jnp.float32),
            pltpu.VMEM((A,), jnp.int32),
            pltpu.VMEM((A,), jnp.float32),
        ],
        compiler_params=pltpu.CompilerParams(needs_layout_passes=False),
    )


@jax.jit
def kernel(signal_features, W, b):
    scores = pl.pallas_call(
        _score_body,
        grid=(B // B_TILE,),
        in_specs=[
            pl.BlockSpec((B_TILE, A, D), lambda i: (i, 0, 0)),
            pl.BlockSpec((D, KM1), lambda i: (0, 0)),
            pl.BlockSpec((KM1, 1), lambda i: (0, 0)),
        ],
        out_specs=pl.BlockSpec((B_TILE, 1, A), lambda i: (i, 0, 0)),
        out_shape=jax.ShapeDtypeStruct((B, 1, A), jnp.float32),
    )(signal_features, W, b.reshape(KM1, 1))
    return _make_sc_select()(scores)
